# baseline (speedup 1.0000x reference)
import functools

import jax
import jax.numpy as jnp
from jax import lax
from jax.experimental import pallas as pl
from jax.experimental.pallas import tpu as pltpu
from jax.experimental.pallas import tpu_sc as plsc

VOCAB = 1000000
EMB_DIM = 64
K_CODES = 1024
B, L = 1024, 200
N = B * L
PAIR = 2 * EMB_DIM


GATHER_WINDOW = 128


def _sc_gather_body(i_vmem, o_vmem, table_hbm):
    pltpu.sync_copy(table_hbm.at[i_vmem.at[0]], o_vmem)


def _sc_gather(table_pairs, idx2d):
    mesh = plsc.VectorSubcoreMesh(core_axis_name="core", subcore_axis_name="subcore")

    @functools.partial(
        pl.kernel,
        out_type=jax.ShapeDtypeStruct((N, PAIR), jnp.float32),
        mesh=mesh,
    )
    def gather_kernel(table_hbm, i_hbm, o_hbm):
        pltpu.emit_pipeline(
            functools.partial(_sc_gather_body, table_hbm=table_hbm),
            grid=(N // GATHER_WINDOW,),
            in_specs=[
                pl.BlockSpec((1, GATHER_WINDOW), index_map=lambda i: (0, i)),
            ],
            out_specs=[
                pl.BlockSpec((GATHER_WINDOW, PAIR), index_map=lambda i: (i, 0)),
            ],
            core_axis_name=("core", "subcore"),
            dimension_semantics=(pltpu.PARALLEL,),
        )(i_hbm, o_hbm)

    return gather_kernel(table_pairs, idx2d)


BLK = 1024
NB = N // BLK


def _vq_body(emb2_ref, par_ref, cb_ref, emb_ref, zq_ref, tok_ref, loss_ref):
    i = pl.program_id(0)
    pair = emb2_ref[...]
    parity = par_ref[...]
    left = pair[:, :EMB_DIM]
    right = pair[:, EMB_DIM:]
    f = jnp.where(parity == 1, right, left)
    emb_ref[...] = f
    cb = cb_ref[...]
    cnorm = jnp.sum(cb * cb, axis=1)
    fnorm = jnp.sum(f * f, axis=1, keepdims=True)
    s = lax.dot_general(f, cb, (((1,), (1,)), ((), ())))
    d = fnorm - 2.0 * s + cnorm[None, :]
    minval = jnp.min(d, axis=1, keepdims=True)
    iota = lax.broadcasted_iota(jnp.int32, d.shape, 1)
    tok = jnp.min(jnp.where(d == minval, iota, K_CODES),
                  axis=1, keepdims=True)
    tok_ref[...] = tok
    onehot = (iota == tok).astype(jnp.float32)
    zq = lax.dot_general(onehot, cb, (((1,), (0,)), ((), ())),
                         precision=lax.Precision.HIGHEST)
    zq_ref[...] = zq
    diff = zq - f
    part = jnp.sum(diff * diff)

    @pl.when(i == 0)
    def _():
        loss_ref[0, 0] = 0.0

    loss_ref[0, 0] += part


def _vq(emb2, parity, codebook):
    emb, zq, tok, loss = pl.pallas_call(
        _vq_body,
        grid=(NB,),
        in_specs=[
            pl.BlockSpec((BLK, PAIR), lambda i: (i, 0)),
            pl.BlockSpec((BLK, 1), lambda i: (i, 0)),
            pl.BlockSpec((K_CODES, EMB_DIM), lambda i: (0, 0)),
        ],
        out_specs=[
            pl.BlockSpec((BLK, EMB_DIM), lambda i: (i, 0)),
            pl.BlockSpec((BLK, EMB_DIM), lambda i: (i, 0)),
            pl.BlockSpec((BLK, 1), lambda i: (i, 0)),
            pl.BlockSpec(memory_space=pltpu.SMEM),
        ],
        out_shape=[
            jax.ShapeDtypeStruct((N, EMB_DIM), jnp.float32),
            jax.ShapeDtypeStruct((N, EMB_DIM), jnp.float32),
            jax.ShapeDtypeStruct((N, 1), jnp.int32),
            jax.ShapeDtypeStruct((1, 1), jnp.float32),
        ],
    )(emb2, parity, codebook)
    return emb, zq, tok, loss


def kernel(x, table, codebook):
    xflat = x.reshape(N).astype(jnp.int32)
    table_pairs = table.reshape(VOCAB // 2, PAIR)
    idx2d = (xflat // 2).reshape(1, N)
    emb2 = _sc_gather(table_pairs, idx2d)
    parity = (xflat & 1).reshape(N, 1)
    emb_flat, zq, tok, loss = _vq(emb2, parity, codebook)
    z_q_st = zq.reshape(B, L, EMB_DIM)
    emb = emb_flat.reshape(B, L, EMB_DIM)
    tokens = tok.reshape(B, L)
    mean_loss = loss[0, 0] / jnp.float32(N * EMB_DIM)
    return (z_q_st, tokens, emb, mean_loss, mean_loss)


# speedup vs baseline: 1.3558x; 1.3558x over previous
import functools

import jax
import jax.numpy as jnp
from jax import lax
from jax.experimental import pallas as pl
from jax.experimental.pallas import tpu as pltpu
from jax.experimental.pallas import tpu_sc as plsc

VOCAB = 1000000
EMB_DIM = 64
K_CODES = 1024
B, L = 1024, 200
N = B * L
PAIR = 2 * EMB_DIM


GATHER_WINDOW = 128


def _sc_gather_body(i_vmem, o_vmem, table_hbm):
    pltpu.sync_copy(table_hbm.at[i_vmem.at[0]], o_vmem)


def _sc_gather(table_pairs, idx2d):
    mesh = plsc.VectorSubcoreMesh(core_axis_name="core", subcore_axis_name="subcore")

    @functools.partial(
        pl.kernel,
        out_type=jax.ShapeDtypeStruct((N, PAIR), jnp.float32),
        mesh=mesh,
    )
    def gather_kernel(table_hbm, i_hbm, o_hbm):
        pltpu.emit_pipeline(
            functools.partial(_sc_gather_body, table_hbm=table_hbm),
            grid=(N // GATHER_WINDOW,),
            in_specs=[
                pl.BlockSpec((1, GATHER_WINDOW), index_map=lambda i: (0, i)),
            ],
            out_specs=[
                pl.BlockSpec((GATHER_WINDOW, PAIR), index_map=lambda i: (i, 0)),
            ],
            core_axis_name=("core", "subcore"),
            dimension_semantics=(pltpu.PARALLEL,),
        )(i_hbm, o_hbm)

    return gather_kernel(table_pairs, idx2d)


BLK = 1024
NB = N // BLK


CHUNK = 128
NCH = K_CODES // CHUNK


def _vq_body(emb2_ref, par_ref, cb_ref, emb_ref, zq_ref, tok_ref, loss_ref,
             cn_ref):
    i = pl.program_id(0)
    cb = cb_ref[...]

    @pl.when(i == 0)
    def _():
        cn_ref[...] = jnp.sum(cb * cb, axis=1)[None, :]
        loss_ref[0, 0] = 0.0

    pair = emb2_ref[...]
    parity = par_ref[...]
    left = pair[:, :EMB_DIM]
    right = pair[:, EMB_DIM:]
    f = jnp.where(parity == 1, right, left)
    emb_ref[...] = f
    fnorm = jnp.sum(f * f, axis=1, keepdims=True)
    s2 = lax.dot_general(-2.0 * f, cb, (((1,), (1,)), ((), ())))
    cnorm = cn_ref[...]
    lane = lax.broadcasted_iota(jnp.int32, (BLK, CHUNK), 1)
    best_v = (fnorm + s2[:, :CHUNK]) + cnorm[:, :CHUNK]
    best_k = lane
    for c in range(1, NCH):
        lo, hi = c * CHUNK, (c + 1) * CHUNK
        d_c = (fnorm + s2[:, lo:hi]) + cnorm[:, lo:hi]
        lt = d_c < best_v
        best_v = jnp.where(lt, d_c, best_v)
        best_k = jnp.where(lt, lane + c * CHUNK, best_k)
    m = jnp.min(best_v, axis=1, keepdims=True)
    tok = jnp.min(jnp.where(best_v == m, best_k, K_CODES),
                  axis=1, keepdims=True)
    tok_ref[...] = tok
    iota = lax.broadcasted_iota(jnp.int32, (BLK, K_CODES), 1)
    onehot = (iota == tok).astype(jnp.float32)
    zq = lax.dot_general(onehot, cb, (((1,), (0,)), ((), ())))
    zq_ref[...] = zq
    diff = zq - f
    loss_ref[0, 0] += jnp.sum(diff * diff)


def _vq(emb2, parity, codebook):
    emb, zq, tok, loss = pl.pallas_call(
        _vq_body,
        grid=(NB,),
        in_specs=[
            pl.BlockSpec((BLK, PAIR), lambda i: (i, 0)),
            pl.BlockSpec((BLK, 1), lambda i: (i, 0)),
            pl.BlockSpec((K_CODES, EMB_DIM), lambda i: (0, 0)),
        ],
        out_specs=[
            pl.BlockSpec((BLK, EMB_DIM), lambda i: (i, 0)),
            pl.BlockSpec((BLK, EMB_DIM), lambda i: (i, 0)),
            pl.BlockSpec((BLK, 1), lambda i: (i, 0)),
            pl.BlockSpec(memory_space=pltpu.SMEM),
        ],
        out_shape=[
            jax.ShapeDtypeStruct((N, EMB_DIM), jnp.float32),
            jax.ShapeDtypeStruct((N, EMB_DIM), jnp.float32),
            jax.ShapeDtypeStruct((N, 1), jnp.int32),
            jax.ShapeDtypeStruct((1, 1), jnp.float32),
        ],
        scratch_shapes=[pltpu.VMEM((1, K_CODES), jnp.float32)],
    )(emb2, parity, codebook)
    return emb, zq, tok, loss


def kernel(x, table, codebook):
    xflat = x.reshape(N).astype(jnp.int32)
    table_pairs = table.reshape(VOCAB // 2, PAIR)
    idx2d = (xflat // 2).reshape(1, N)
    emb2 = _sc_gather(table_pairs, idx2d)
    parity = (xflat & 1).reshape(N, 1)
    emb_flat, zq, tok, loss = _vq(emb2, parity, codebook)
    z_q_st = zq.reshape(B, L, EMB_DIM)
    emb = emb_flat.reshape(B, L, EMB_DIM)
    tokens = tok.reshape(B, L)
    mean_loss = loss[0, 0] / jnp.float32(N * EMB_DIM)
    return (z_q_st, tokens, emb, mean_loss, mean_loss)
